# speedup vs baseline: 3.0444x; 3.0444x over previous
import functools

import jax
import jax.numpy as jnp
from jax import lax
from jax.experimental import pallas as pl
from jax.experimental.pallas import tpu as pltpu
from jax.experimental.pallas import tpu_sc as plsc

_D = 128
_CHUNK = 128


@functools.cache
def _build(B):
    info = plsc.get_sparse_core_info()
    nc, ns = info.num_cores, info.num_subcores
    nw = nc * ns
    b_per_w = B // nw
    n_chunks = b_per_w // _CHUNK

    mesh = plsc.VectorSubcoreMesh(core_axis_name="c", subcore_axis_name="s")

    @functools.partial(
        pl.kernel,
        out_type=jax.ShapeDtypeStruct((B, _D), jnp.float32),
        mesh=mesh,
        scratch_types=[
            pltpu.VMEM((n_chunks, _CHUNK), jnp.int32),
            pltpu.VMEM((_CHUNK, _D), jnp.float32),
            pltpu.SemaphoreType.DMA,
        ],
    )
    def gather_kernel(pos_hbm, table_hbm, out_hbm, idx_v, rows_v, sem):
        wid = lax.axis_index("s") * nc + lax.axis_index("c")
        base = wid * b_per_w
        pltpu.sync_copy(pos_hbm.at[pl.ds(wid * n_chunks, n_chunks)], idx_v)

        def step(j, carry):
            pltpu.async_copy(table_hbm.at[idx_v.at[j]], rows_v, sem).wait()
            pltpu.sync_copy(rows_v, out_hbm.at[pl.ds(base + j * _CHUNK, _CHUNK)])
            return carry

        lax.fori_loop(0, n_chunks, step, 0)

    return gather_kernel


def kernel(position, table):
    b0, b1 = position.shape
    B = b0 * b1
    pos2d = position.reshape(B // _CHUNK, _CHUNK).astype(jnp.int32)
    out = _build(B)(pos2d, table)
    return out.reshape(b0, b1, _D)


# speedup vs baseline: 3.4555x; 1.1350x over previous
import functools

import jax
import jax.numpy as jnp
from jax import lax
from jax.experimental import pallas as pl
from jax.experimental.pallas import tpu as pltpu
from jax.experimental.pallas import tpu_sc as plsc

_D = 128
_CHUNK = 128
_NBUF = 4


@functools.cache
def _build(B):
    info = plsc.get_sparse_core_info()
    nc, ns = info.num_cores, info.num_subcores
    nw = nc * ns
    b_per_w = B // nw
    n_chunks = b_per_w // _CHUNK
    n_groups = n_chunks // _NBUF

    mesh = plsc.VectorSubcoreMesh(core_axis_name="c", subcore_axis_name="s")

    @functools.partial(
        pl.kernel,
        out_type=jax.ShapeDtypeStruct((B, _D), jnp.float32),
        mesh=mesh,
        scratch_types=[
            pltpu.VMEM((n_chunks, _CHUNK), jnp.int32),
            pltpu.VMEM((_NBUF, _CHUNK, _D), jnp.float32),
        ]
        + [pltpu.SemaphoreType.DMA] * (2 * _NBUF),
    )
    def gather_kernel(pos_hbm, table_hbm, out_hbm, idx_v, rows_v, *sems):
        gsem, ssem = sems[:_NBUF], sems[_NBUF:]
        wid = lax.axis_index("s") * nc + lax.axis_index("c")
        base = wid * b_per_w
        pltpu.sync_copy(pos_hbm.at[pl.ds(wid * n_chunks, n_chunks)], idx_v)

        def fire_gather(j, b):
            pltpu.async_copy(table_hbm.at[idx_v.at[j]], rows_v.at[b], gsem[b])

        def wait_gather(b):
            pltpu.make_async_copy(
                table_hbm.at[idx_v.at[b]], rows_v.at[b], gsem[b]
            ).wait()

        def fire_store(j, b):
            pltpu.async_copy(
                rows_v.at[b], out_hbm.at[pl.ds(base + j * _CHUNK, _CHUNK)], ssem[b]
            )

        def wait_store(j, b):
            pltpu.make_async_copy(
                rows_v.at[b], out_hbm.at[pl.ds(base + j * _CHUNK, _CHUNK)], ssem[b]
            ).wait()

        for b in range(_NBUF):
            fire_gather(b, b)

        def group(g, carry):
            j0 = g * _NBUF
            for b in range(_NBUF):
                wait_gather(b)
                fire_store(j0 + b, b)
            for b in range(_NBUF):
                wait_store(j0 + b, b)
                fire_gather(j0 + b + _NBUF, b)
            return carry

        lax.fori_loop(0, n_groups - 1, group, 0)

        j0 = (n_groups - 1) * _NBUF
        for b in range(_NBUF):
            wait_gather(b)
            fire_store(j0 + b, b)
        for b in range(_NBUF):
            wait_store(j0 + b, b)

    return gather_kernel


def kernel(position, table):
    b0, b1 = position.shape
    B = b0 * b1
    pos2d = position.reshape(B // _CHUNK, _CHUNK).astype(jnp.int32)
    out = _build(B)(pos2d, table)
    return out.reshape(b0, b1, _D)


# speedup vs baseline: 11.7547x; 3.4018x over previous
import functools

import jax
import jax.numpy as jnp
from jax import lax
from jax.experimental import pallas as pl
from jax.experimental.pallas import tpu as pltpu
from jax.experimental.pallas import tpu_sc as plsc

_D = 128
_CHUNK = 128
_NBUF = 4


@functools.cache
def _build(B):
    info = plsc.get_sparse_core_info()
    nc, ns = info.num_cores, info.num_subcores
    nw = nc * ns
    b_per_w = B // nw
    n_chunks = b_per_w // _CHUNK
    n_groups = n_chunks // _NBUF

    mesh = plsc.VectorSubcoreMesh(core_axis_name="c", subcore_axis_name="s")

    @functools.partial(
        pl.kernel,
        out_type=jax.ShapeDtypeStruct((B, _D), jnp.float32),
        mesh=mesh,
        scratch_types=[
            pltpu.VMEM((n_chunks, _CHUNK), jnp.int32),
            pltpu.VMEM((_NBUF, _CHUNK, _D), jnp.float32),
        ]
        + [pltpu.SemaphoreType.DMA] * (2 * _NBUF),
    )
    def gather_kernel(pos_hbm, table_hbm, out_hbm, idx_v, rows_v, *sems):
        gsem, ssem = sems[:_NBUF], sems[_NBUF:]
        wid = lax.axis_index("s") * nc + lax.axis_index("c")
        base = wid * b_per_w
        pltpu.sync_copy(pos_hbm.at[pl.ds(wid * n_chunks, n_chunks)], idx_v)

        def fire_gather(j, b):
            pltpu.async_copy(table_hbm.at[idx_v.at[j]], rows_v.at[b], gsem[b])

        def wait_gather(b):
            pltpu.make_async_copy(
                table_hbm.at[idx_v.at[b]], rows_v.at[b], gsem[b]
            ).wait()

        def fire_store(j, b):
            pltpu.async_copy(
                rows_v.at[b], out_hbm.at[pl.ds(base + j * _CHUNK, _CHUNK)], ssem[b]
            )

        def wait_store(j, b):
            pltpu.make_async_copy(
                rows_v.at[b], out_hbm.at[pl.ds(base + j * _CHUNK, _CHUNK)], ssem[b]
            ).wait()

        for b in range(_NBUF):
            fire_gather(b, b)

        def group(g, carry):
            j0 = g * _NBUF
            for b in range(_NBUF):
                wait_gather(b)
                fire_store(j0 + b, b)
            for b in range(_NBUF):
                wait_store(j0 + b, b)
                fire_gather(j0 + b + _NBUF, b)
            return carry

        lax.fori_loop(0, n_groups - 1, group, 0)

        j0 = (n_groups - 1) * _NBUF
        for b in range(_NBUF):
            wait_gather(b)
            fire_store(j0 + b, b)
        for b in range(_NBUF):
            wait_store(j0 + b, b)

    return gather_kernel


def kernel(position, table):
    b0, b1 = position.shape
    B = b0 * b1
    pos2d = position.T.reshape(B // _CHUNK, _CHUNK).astype(jnp.int32)
    out = _build(B)(pos2d, table)
    return out.reshape(b1, b0, _D).transpose(1, 0, 2)


# speedup vs baseline: 11.7763x; 1.0018x over previous
import functools

import jax
import jax.numpy as jnp
from jax import lax
from jax.experimental import pallas as pl
from jax.experimental.pallas import tpu as pltpu
from jax.experimental.pallas import tpu_sc as plsc

_D = 128
_CHUNK = 128
_NBUF = 5


@functools.cache
def _build(B):
    info = plsc.get_sparse_core_info()
    nc, ns = info.num_cores, info.num_subcores
    nw = nc * ns
    b_per_w = B // nw
    n_chunks = b_per_w // _CHUNK
    n_groups = n_chunks // _NBUF

    mesh = plsc.VectorSubcoreMesh(core_axis_name="c", subcore_axis_name="s")

    @functools.partial(
        pl.kernel,
        out_type=jax.ShapeDtypeStruct((B, _D), jnp.float32),
        mesh=mesh,
        scratch_types=[
            pltpu.VMEM((n_chunks, _CHUNK), jnp.int32),
            pltpu.VMEM((_NBUF, _CHUNK, _D), jnp.float32),
        ]
        + [pltpu.SemaphoreType.DMA] * (2 * _NBUF),
    )
    def gather_kernel(pos_hbm, table_hbm, out_hbm, idx_v, rows_v, *sems):
        gsem, ssem = sems[:_NBUF], sems[_NBUF:]
        wid = lax.axis_index("s") * nc + lax.axis_index("c")
        base = wid * b_per_w
        pltpu.sync_copy(pos_hbm.at[pl.ds(wid * n_chunks, n_chunks)], idx_v)

        def fire_gather(j, b):
            pltpu.async_copy(table_hbm.at[idx_v.at[j]], rows_v.at[b], gsem[b])

        def wait_gather(b):
            pltpu.make_async_copy(
                table_hbm.at[idx_v.at[b]], rows_v.at[b], gsem[b]
            ).wait()

        def fire_store(j, b):
            pltpu.async_copy(
                rows_v.at[b], out_hbm.at[pl.ds(base + j * _CHUNK, _CHUNK)], ssem[b]
            )

        def wait_store(j, b):
            pltpu.make_async_copy(
                rows_v.at[b], out_hbm.at[pl.ds(base + j * _CHUNK, _CHUNK)], ssem[b]
            ).wait()

        for b in range(_NBUF):
            fire_gather(b, b)

        def group(g, carry):
            j0 = g * _NBUF
            for b in range(_NBUF):
                wait_gather(b)
                fire_store(j0 + b, b)
            for b in range(_NBUF):
                wait_store(j0 + b, b)
                fire_gather(j0 + b + _NBUF, b)
            return carry

        lax.fori_loop(0, n_groups - 1, group, 0)

        j0 = (n_groups - 1) * _NBUF
        for b in range(_NBUF):
            wait_gather(b)
            fire_store(j0 + b, b)
        for b in range(_NBUF):
            wait_store(j0 + b, b)

    return gather_kernel


def kernel(position, table):
    b0, b1 = position.shape
    B = b0 * b1
    pos2d = position.T.reshape(B // _CHUNK, _CHUNK).astype(jnp.int32)
    out = _build(B)(pos2d, table)
    return out.reshape(b1, b0, _D).transpose(1, 0, 2)


# speedup vs baseline: 11.8107x; 1.0029x over previous
import functools

import jax
import jax.numpy as jnp
from jax import lax
from jax.experimental import pallas as pl
from jax.experimental.pallas import tpu as pltpu
from jax.experimental.pallas import tpu_sc as plsc

_D = 128
_CHUNK = 128
_R = 2


@functools.cache
def _build(B):
    info = plsc.get_sparse_core_info()
    nc, ns = info.num_cores, info.num_subcores
    nw = nc * ns
    b_per_w = B // nw
    n_chunks = b_per_w // _CHUNK
    n_groups = n_chunks // _R
    assert n_groups % 2 == 0 and n_groups >= 4

    mesh = plsc.VectorSubcoreMesh(core_axis_name="c", subcore_axis_name="s")

    @functools.partial(
        pl.kernel,
        out_type=jax.ShapeDtypeStruct((B, _D), jnp.float32),
        mesh=mesh,
        scratch_types=[
            pltpu.VMEM((n_chunks, _CHUNK), jnp.int32),
            pltpu.VMEM((2, _R, _CHUNK, _D), jnp.float32),
        ]
        + [pltpu.SemaphoreType.DMA] * (4 * _R),
    )
    def gather_kernel(pos_hbm, table_hbm, out_hbm, idx_v, rows_v, *sems):
        gsem = (sems[0:_R], sems[_R : 2 * _R])
        ssem = (sems[2 * _R : 3 * _R], sems[3 * _R : 4 * _R])
        wid = lax.axis_index("s") * nc + lax.axis_index("c")
        base = wid * b_per_w
        pltpu.sync_copy(pos_hbm.at[pl.ds(wid * n_chunks, n_chunks)], idx_v)

        def fire_gather(g, a, b):
            pltpu.async_copy(
                table_hbm.at[idx_v.at[g * _R + b]], rows_v.at[a, b], gsem[a][b]
            )

        def wait_gather(a, b):
            pltpu.make_async_copy(
                table_hbm.at[idx_v.at[b]], rows_v.at[a, b], gsem[a][b]
            ).wait()

        def fire_store(g, a, b):
            pltpu.async_copy(
                rows_v.at[a, b],
                out_hbm.at[pl.ds(base + (g * _R + b) * _CHUNK, _CHUNK)],
                ssem[a][b],
            )

        def wait_store(g, a, b):
            pltpu.make_async_copy(
                rows_v.at[a, b],
                out_hbm.at[pl.ds(base + (g * _R + b) * _CHUNK, _CHUNK)],
                ssem[a][b],
            ).wait()

        for b in range(_R):
            fire_gather(0, 0, b)
        for b in range(_R):
            wait_gather(0, b)
        for b in range(_R):
            fire_store(0, 0, b)
        for b in range(_R):
            fire_gather(1, 1, b)

        def pair(t, carry):
            g_odd = 1 + 2 * t
            for b in range(_R):
                wait_gather(1, b)
            for b in range(_R):
                fire_store(g_odd, 1, b)
            for b in range(_R):
                wait_store(g_odd - 1, 0, b)
            for b in range(_R):
                fire_gather(g_odd + 1, 0, b)
            g_even = 2 + 2 * t
            for b in range(_R):
                wait_gather(0, b)
            for b in range(_R):
                fire_store(g_even, 0, b)
            for b in range(_R):
                wait_store(g_even - 1, 1, b)
            for b in range(_R):
                fire_gather(g_even + 1, 1, b)
            return carry

        lax.fori_loop(0, (n_groups - 2) // 2, pair, 0)

        g_last = n_groups - 1
        for b in range(_R):
            wait_gather(1, b)
        for b in range(_R):
            fire_store(g_last, 1, b)
        for b in range(_R):
            wait_store(g_last - 1, 0, b)
        for b in range(_R):
            wait_store(g_last, 1, b)

    return gather_kernel


def kernel(position, table):
    b0, b1 = position.shape
    B = b0 * b1
    pos2d = position.T.reshape(B // _CHUNK, _CHUNK).astype(jnp.int32)
    out = _build(B)(pos2d, table)
    return out.reshape(b1, b0, _D).transpose(1, 0, 2)


# speedup vs baseline: 11.9118x; 1.0086x over previous
import functools

import jax
import jax.numpy as jnp
from jax import lax
from jax.experimental import pallas as pl
from jax.experimental.pallas import tpu as pltpu
from jax.experimental.pallas import tpu_sc as plsc

_D = 128
_CHUNK = 256
_R = 1


@functools.cache
def _build(B):
    info = plsc.get_sparse_core_info()
    nc, ns = info.num_cores, info.num_subcores
    nw = nc * ns
    b_per_w = B // nw
    n_chunks = b_per_w // _CHUNK
    n_groups = n_chunks // _R
    assert n_groups % 2 == 0 and n_groups >= 4

    mesh = plsc.VectorSubcoreMesh(core_axis_name="c", subcore_axis_name="s")

    @functools.partial(
        pl.kernel,
        out_type=jax.ShapeDtypeStruct((B, _D), jnp.float32),
        mesh=mesh,
        scratch_types=[
            pltpu.VMEM((b_per_w,), jnp.int32),
            pltpu.VMEM((2, _R, _CHUNK, _D), jnp.float32),
        ]
        + [pltpu.SemaphoreType.DMA] * (4 * _R),
    )
    def gather_kernel(pos_hbm, table_hbm, out_hbm, idx_v, rows_v, *sems):
        gsem = (sems[0:_R], sems[_R : 2 * _R])
        ssem = (sems[2 * _R : 3 * _R], sems[3 * _R : 4 * _R])
        wid = lax.axis_index("s") * nc + lax.axis_index("c")
        base = wid * b_per_w
        pltpu.sync_copy(pos_hbm.at[pl.ds(wid * b_per_w, b_per_w)], idx_v)

        def fire_gather(g, a, b):
            pltpu.async_copy(
                table_hbm.at[idx_v.at[pl.ds((g * _R + b) * _CHUNK, _CHUNK)]],
                rows_v.at[a, b],
                gsem[a][b],
            )

        def wait_gather(a, b):
            pltpu.make_async_copy(
                table_hbm.at[idx_v.at[pl.ds(b * _CHUNK, _CHUNK)]], rows_v.at[a, b], gsem[a][b]
            ).wait()

        def fire_store(g, a, b):
            pltpu.async_copy(
                rows_v.at[a, b],
                out_hbm.at[pl.ds(base + (g * _R + b) * _CHUNK, _CHUNK)],
                ssem[a][b],
            )

        def wait_store(g, a, b):
            pltpu.make_async_copy(
                rows_v.at[a, b],
                out_hbm.at[pl.ds(base + (g * _R + b) * _CHUNK, _CHUNK)],
                ssem[a][b],
            ).wait()

        for b in range(_R):
            fire_gather(0, 0, b)
        for b in range(_R):
            wait_gather(0, b)
        for b in range(_R):
            fire_store(0, 0, b)
        for b in range(_R):
            fire_gather(1, 1, b)

        def pair(t, carry):
            g_odd = 1 + 2 * t
            for b in range(_R):
                wait_gather(1, b)
            for b in range(_R):
                fire_store(g_odd, 1, b)
            for b in range(_R):
                wait_store(g_odd - 1, 0, b)
            for b in range(_R):
                fire_gather(g_odd + 1, 0, b)
            g_even = 2 + 2 * t
            for b in range(_R):
                wait_gather(0, b)
            for b in range(_R):
                fire_store(g_even, 0, b)
            for b in range(_R):
                wait_store(g_even - 1, 1, b)
            for b in range(_R):
                fire_gather(g_even + 1, 1, b)
            return carry

        lax.fori_loop(0, (n_groups - 2) // 2, pair, 0)

        g_last = n_groups - 1
        for b in range(_R):
            wait_gather(1, b)
        for b in range(_R):
            fire_store(g_last, 1, b)
        for b in range(_R):
            wait_store(g_last - 1, 0, b)
        for b in range(_R):
            wait_store(g_last, 1, b)

    return gather_kernel


def kernel(position, table):
    b0, b1 = position.shape
    B = b0 * b1
    pos2d = position.T.reshape(B).astype(jnp.int32)
    out = _build(B)(pos2d, table)
    return out.reshape(b1, b0, _D).transpose(1, 0, 2)
